# Initial kernel scaffold; baseline (speedup 1.0000x reference)
import functools

import jax
import jax.numpy as jnp
from jax import lax
from jax.experimental import pallas as pl
from jax.experimental.pallas import tpu as pltpu
from jax.experimental.pallas import tpu_sc as plsc

N_NODES = 10000
N_EDGES = 320000
DIM = 128

NC = 2
NS = 16
NW = NC * NS
K = 128
CHUNKS = -(-N_EDGES // (NW * K))
E_PAD = NW * CHUNKS * K
N_ACC = 10016
RPT = N_ACC // NS
CW = 8

RB = 2000


def _sc_agg_body(x_hbm, src_hbm, dst_hbm, zrow_hbm, zcnt_hbm, ones_hbm,
                 part_hbm, cnt_hbm,
                 sidx_v, didx_v, rows_v, ones_v, acc_sh, cacc_sh, sem):
    c = lax.axis_index("c")
    s = lax.axis_index("s")
    wid = c * NS + s
    r0 = s * RPT
    pltpu.sync_copy(zrow_hbm.at[pl.ds(r0, RPT)], acc_sh.at[pl.ds(r0, RPT)])
    pltpu.sync_copy(zcnt_hbm.at[pl.ds(r0, RPT)], cacc_sh.at[pl.ds(r0, RPT)])
    pltpu.sync_copy(ones_hbm, ones_v)
    plsc.subcore_barrier()

    base = wid * (CHUNKS * K)

    def body(i, carry):
        off = base + i * K
        pltpu.sync_copy(src_hbm.at[pl.ds(off, K)], sidx_v)
        pltpu.sync_copy(dst_hbm.at[pl.ds(off, K)], didx_v)
        pltpu.async_copy(x_hbm.at[sidx_v], rows_v, sem).wait()
        pltpu.sync_copy(rows_v, acc_sh.at[didx_v], add=True)
        pltpu.sync_copy(ones_v, cacc_sh.at[didx_v], add=True)
        return carry

    lax.fori_loop(0, CHUNKS, body, 0)
    plsc.subcore_barrier()
    pltpu.sync_copy(acc_sh.at[pl.ds(r0, RPT)], part_hbm.at[c, pl.ds(r0, RPT)])
    pltpu.sync_copy(cacc_sh.at[pl.ds(r0, RPT)], cnt_hbm.at[c, pl.ds(r0, RPT)])


@functools.cache
def _make_sc_agg():
    mesh = plsc.VectorSubcoreMesh(
        core_axis_name="c", subcore_axis_name="s", num_cores=NC,
        num_subcores=NS)
    return pl.kernel(
        _sc_agg_body,
        out_type=(
            jax.ShapeDtypeStruct((NC, N_ACC, DIM), jnp.float32),
            jax.ShapeDtypeStruct((NC, N_ACC, CW), jnp.float32),
        ),
        mesh=mesh,
        scratch_types=[
            pltpu.VMEM((K,), jnp.int32),
            pltpu.VMEM((K,), jnp.int32),
            pltpu.VMEM((K, DIM), jnp.float32),
            pltpu.VMEM((K, CW), jnp.float32),
            pltpu.VMEM_SHARED((N_ACC, DIM), jnp.float32),
            pltpu.VMEM_SHARED((N_ACC, CW), jnp.float32),
            pltpu.SemaphoreType.DMA,
        ],
        name="sage_segment_sum_sc",
    )


def _tc_layer_kernel(part_ref, cnt_ref, x_ref, wl_ref, wr_ref, b_ref, o_ref):
    cnt = cnt_ref[0, :, 0:1] + cnt_ref[1, :, 0:1]
    recip = 1.0 / jnp.maximum(cnt, 1.0)
    mean = (part_ref[0] + part_ref[1]) * recip
    acc = lax.dot_general(mean, wl_ref[...], (((1,), (1,)), ((), ())),
                          preferred_element_type=jnp.float32)
    acc = acc + lax.dot_general(x_ref[...], wr_ref[...],
                                (((1,), (1,)), ((), ())),
                                preferred_element_type=jnp.float32)
    o_ref[...] = jnp.maximum(acc + b_ref[...], 0.0)


def _tc_layer(part, cnt, x, W_l, W_r, b):
    grid = N_NODES // RB
    return pl.pallas_call(
        _tc_layer_kernel,
        grid=(grid,),
        in_specs=[
            pl.BlockSpec((NC, RB, DIM), lambda i: (0, i, 0)),
            pl.BlockSpec((NC, RB, CW), lambda i: (0, i, 0)),
            pl.BlockSpec((RB, DIM), lambda i: (i, 0)),
            pl.BlockSpec((DIM, DIM), lambda i: (0, 0)),
            pl.BlockSpec((DIM, DIM), lambda i: (0, 0)),
            pl.BlockSpec((1, DIM), lambda i: (0, 0)),
        ],
        out_specs=pl.BlockSpec((RB, DIM), lambda i: (i, 0)),
        out_shape=jax.ShapeDtypeStruct((N_NODES, DIM), jnp.float32),
        name="sage_dense_tc",
    )(part, cnt, x, W_l, W_r, b.reshape(1, DIM))


def kernel(x, edge_index, W1_l, b1_l, W1_r, W2_l, b2_l, W2_r):
    src = edge_index[0].astype(jnp.int32)
    dst = edge_index[1].astype(jnp.int32)
    pad = E_PAD - N_EDGES
    src = jnp.concatenate([src, jnp.zeros((pad,), jnp.int32)])
    dst = jnp.concatenate([dst, jnp.full((pad,), N_NODES, jnp.int32)])
    zrow = jnp.zeros((N_ACC, DIM), jnp.float32)
    zcnt = jnp.zeros((N_ACC, CW), jnp.float32)
    ones = jnp.ones((K, CW), jnp.float32)

    sc_agg = _make_sc_agg()
    part1, cnt = sc_agg(x, src, dst, zrow, zcnt, ones)
    h1 = _tc_layer(part1, cnt, x, W1_l, W1_r, b1_l)
    part2, _ = sc_agg(h1, src, dst, zrow, zcnt, ones)
    h2 = _tc_layer(part2, cnt, h1, W2_l, W2_r, b2_l)
    return h2


# speedup vs baseline: 3.9357x; 3.9357x over previous
import functools

import jax
import jax.numpy as jnp
from jax import lax
from jax.experimental import pallas as pl
from jax.experimental.pallas import tpu as pltpu
from jax.experimental.pallas import tpu_sc as plsc

N_NODES = 10000
N_EDGES = 320000
DIM = 128

NC = 2
NS = 16
NW = NC * NS
K = 128
CHUNKS = -(-N_EDGES // (NW * K))
E_PAD = NW * CHUNKS * K
N_ACC = 10240
RPT = N_ACC // NS
PUB = RPT // K

RB = 2000


def _sc_sum_body(x_hbm, src_hbm, dst_hbm, iota_hbm, zrow_hbm,
                 part_hbm,
                 sidx_v, didx_v, rows_v, acc_sh, sem):
    c = lax.axis_index("c")
    s = lax.axis_index("s")
    wid = c * NS + s
    r0 = s * RPT
    pltpu.sync_copy(zrow_hbm, rows_v)
    for j in range(PUB):
        pltpu.sync_copy(iota_hbm.at[pl.ds(r0 + j * K, K)], sidx_v)
        pltpu.sync_copy(rows_v, acc_sh.at[sidx_v])
    plsc.subcore_barrier()

    base = wid * (CHUNKS * K)

    def body(i, carry):
        off = base + i * K
        pltpu.sync_copy(src_hbm.at[pl.ds(off, K)], sidx_v)
        pltpu.sync_copy(dst_hbm.at[pl.ds(off, K)], didx_v)
        pltpu.async_copy(x_hbm.at[sidx_v], rows_v, sem).wait()
        pltpu.sync_copy(rows_v, acc_sh.at[didx_v], add=True)
        return carry

    lax.fori_loop(0, CHUNKS, body, 0)
    plsc.subcore_barrier()
    for j in range(PUB):
        pltpu.sync_copy(iota_hbm.at[pl.ds(r0 + j * K, K)], sidx_v)
        pltpu.sync_copy(acc_sh.at[sidx_v], rows_v)
        pltpu.sync_copy(rows_v, part_hbm.at[c, pl.ds(r0 + j * K, K)])


def _sc_cnt_body(dst_hbm, iota_hbm, zrow_hbm, ones_hbm,
                 cnt_hbm,
                 sidx_v, didx_v, rows_v, ones_v, acc_sh, sem):
    c = lax.axis_index("c")
    s = lax.axis_index("s")
    wid = c * NS + s
    r0 = s * RPT
    pltpu.sync_copy(zrow_hbm, rows_v)
    pltpu.sync_copy(ones_hbm, ones_v)
    for j in range(PUB):
        pltpu.sync_copy(iota_hbm.at[pl.ds(r0 + j * K, K)], sidx_v)
        pltpu.sync_copy(rows_v, acc_sh.at[sidx_v])
    plsc.subcore_barrier()

    base = wid * (CHUNKS * K)

    def body(i, carry):
        off = base + i * K
        pltpu.sync_copy(dst_hbm.at[pl.ds(off, K)], didx_v)
        pltpu.sync_copy(ones_v, acc_sh.at[didx_v], add=True)
        return carry

    lax.fori_loop(0, CHUNKS, body, 0)
    plsc.subcore_barrier()
    for j in range(PUB):
        pltpu.sync_copy(iota_hbm.at[pl.ds(r0 + j * K, K)], sidx_v)
        pltpu.sync_copy(acc_sh.at[sidx_v], rows_v)
        pltpu.sync_copy(rows_v, cnt_hbm.at[c, pl.ds(r0 + j * K, K)])


@functools.cache
def _make_mesh():
    return plsc.VectorSubcoreMesh(
        core_axis_name="c", subcore_axis_name="s", num_cores=NC,
        num_subcores=NS)


@functools.cache
def _make_sc_sum():
    return pl.kernel(
        _sc_sum_body,
        out_type=jax.ShapeDtypeStruct((NC, N_ACC, DIM), jnp.float32),
        mesh=_make_mesh(),
        scratch_types=[
            pltpu.VMEM((K,), jnp.int32),
            pltpu.VMEM((K,), jnp.int32),
            pltpu.VMEM((K, DIM), jnp.float32),
            pltpu.VMEM_SHARED((N_ACC, DIM), jnp.float32),
            pltpu.SemaphoreType.DMA,
        ],
        name="sage_segment_sum_sc",
    )


@functools.cache
def _make_sc_cnt():
    return pl.kernel(
        _sc_cnt_body,
        out_type=jax.ShapeDtypeStruct((NC, N_ACC, DIM), jnp.float32),
        mesh=_make_mesh(),
        scratch_types=[
            pltpu.VMEM((K,), jnp.int32),
            pltpu.VMEM((K,), jnp.int32),
            pltpu.VMEM((K, DIM), jnp.float32),
            pltpu.VMEM((K, DIM), jnp.float32),
            pltpu.VMEM_SHARED((N_ACC, DIM), jnp.float32),
            pltpu.SemaphoreType.DMA,
        ],
        name="sage_degree_count_sc",
    )


def _tc_layer_kernel(part_ref, cnt_ref, x_ref, wl_ref, wr_ref, b_ref, o_ref):
    cnt = cnt_ref[0, :, 0:1] + cnt_ref[1, :, 0:1]
    recip = 1.0 / jnp.maximum(cnt, 1.0)
    mean = (part_ref[0] + part_ref[1]) * recip
    acc = lax.dot_general(mean, wl_ref[...], (((1,), (1,)), ((), ())),
                          preferred_element_type=jnp.float32)
    acc = acc + lax.dot_general(x_ref[...], wr_ref[...],
                                (((1,), (1,)), ((), ())),
                                preferred_element_type=jnp.float32)
    o_ref[...] = jnp.maximum(acc + b_ref[...], 0.0)


def _tc_layer(part, cnt, x, W_l, W_r, b):
    grid = N_NODES // RB
    return pl.pallas_call(
        _tc_layer_kernel,
        grid=(grid,),
        in_specs=[
            pl.BlockSpec((NC, RB, DIM), lambda i: (0, i, 0)),
            pl.BlockSpec((NC, RB, DIM), lambda i: (0, i, 0)),
            pl.BlockSpec((RB, DIM), lambda i: (i, 0)),
            pl.BlockSpec((DIM, DIM), lambda i: (0, 0)),
            pl.BlockSpec((DIM, DIM), lambda i: (0, 0)),
            pl.BlockSpec((1, DIM), lambda i: (0, 0)),
        ],
        out_specs=pl.BlockSpec((RB, DIM), lambda i: (i, 0)),
        out_shape=jax.ShapeDtypeStruct((N_NODES, DIM), jnp.float32),
        name="sage_dense_tc",
    )(part, cnt, x, W_l, W_r, b.reshape(1, DIM))


def kernel(x, edge_index, W1_l, b1_l, W1_r, W2_l, b2_l, W2_r):
    src = edge_index[0].astype(jnp.int32)
    dst = edge_index[1].astype(jnp.int32)
    pad = E_PAD - N_EDGES
    src = jnp.concatenate([src, jnp.zeros((pad,), jnp.int32)])
    dst = jnp.concatenate([dst, jnp.full((pad,), N_NODES, jnp.int32)])
    iota = jnp.arange(N_ACC, dtype=jnp.int32)
    zrow = jnp.zeros((K, DIM), jnp.float32)
    ones = jnp.ones((K, DIM), jnp.float32)

    cnt = _make_sc_cnt()(dst, iota, zrow, ones)
    part1 = _make_sc_sum()(x, src, dst, iota, zrow)
    h1 = _tc_layer(part1, cnt, x, W1_l, W1_r, b1_l)
    part2 = _make_sc_sum()(h1, src, dst, iota, zrow)
    h2 = _tc_layer(part2, cnt, h1, W2_l, W2_r, b2_l)
    return h2
